# baseline (speedup 1.0000x reference)
import functools

import jax
import jax.numpy as jnp
from jax import lax
from jax.experimental import pallas as pl
from jax.experimental.pallas import tpu as pltpu
from jax.experimental.pallas import tpu_sc as plsc

N = 10000
E = 320000
IN_DIM = 128
H = 8
D = 16
HD = H * D
KV_W = 2 * HD
ACC_W = 144

NC = 2
NS = 16
NW = NC * NS
EW = E // NW
B = 32
NBLK = 2 * -(-EW // (2 * B))
EWP = NBLK * B
E_PAD = NW * EWP
PAD_DST = 10100
RPT = 632
N_PAD = NS * RPT
ZROWS = 640
ZPT = ZROWS // NS
CH = 128
NCHK = N_PAD // CH

_ROWS_PER_PROJ_BLOCK = 1000


def _proj_body(h_ref, wq_ref, wk_ref, wv_ref, bq_ref, bk_ref, bv_ref,
               q_ref, kv_ref):
    hb = h_ref[...]
    q = jnp.dot(hb, wq_ref[...], preferred_element_type=jnp.float32) + bq_ref[...]
    q_ref[...] = q * 0.25
    kv_ref[:, :HD] = jnp.dot(hb, wk_ref[...], preferred_element_type=jnp.float32) + bk_ref[...]
    kv_ref[:, HD:] = jnp.dot(hb, wv_ref[...], preferred_element_type=jnp.float32) + bv_ref[...]


_proj = pl.pallas_call(
    _proj_body,
    grid=(N // _ROWS_PER_PROJ_BLOCK,),
    in_specs=[
        pl.BlockSpec((_ROWS_PER_PROJ_BLOCK, IN_DIM), lambda i: (i, 0)),
        pl.BlockSpec((IN_DIM, HD), lambda i: (0, 0)),
        pl.BlockSpec((IN_DIM, HD), lambda i: (0, 0)),
        pl.BlockSpec((IN_DIM, HD), lambda i: (0, 0)),
        pl.BlockSpec((1, HD), lambda i: (0, 0)),
        pl.BlockSpec((1, HD), lambda i: (0, 0)),
        pl.BlockSpec((1, HD), lambda i: (0, 0)),
    ],
    out_specs=[
        pl.BlockSpec((_ROWS_PER_PROJ_BLOCK, HD), lambda i: (i, 0)),
        pl.BlockSpec((_ROWS_PER_PROJ_BLOCK, KV_W), lambda i: (i, 0)),
    ],
    out_shape=[
        jax.ShapeDtypeStruct((N, HD), jnp.float32),
        jax.ShapeDtypeStruct((N, KV_W), jnp.float32),
    ],
)

_mesh = plsc.VectorSubcoreMesh(core_axis_name="c", subcore_axis_name="s")
_sc_params = pltpu.CompilerParams(needs_layout_passes=False)


@functools.partial(
    pl.kernel,
    out_type=(
        jax.ShapeDtypeStruct((NC * N_PAD, HD), jnp.float32),
        jax.ShapeDtypeStruct((NC * ZROWS, HD), jnp.float32),
    ),
    mesh=_mesh,
    scratch_types=[
        pltpu.VMEM((B,), jnp.int32),
        pltpu.VMEM((B,), jnp.int32),
        pltpu.VMEM((B,), jnp.int32),
        pltpu.VMEM((B,), jnp.int32),
        pltpu.VMEM((B,), jnp.int32),
        pltpu.VMEM((B, KV_W), jnp.float32),
        pltpu.VMEM((B, HD), jnp.float32),
        pltpu.VMEM((B, KV_W), jnp.float32),
        pltpu.VMEM((B, HD), jnp.float32),
        pltpu.VMEM((B, HD), jnp.float32),
        pltpu.VMEM((B, HD), jnp.float32),
        pltpu.VMEM((B, D), jnp.float32),
        pltpu.VMEM_SHARED((N_PAD, HD), jnp.float32),
        pltpu.VMEM_SHARED((ZROWS, HD), jnp.float32),
        pltpu.SemaphoreType.DMA,
        pltpu.SemaphoreType.DMA,
        pltpu.SemaphoreType.DMA,
    ],
    compiler_params=_sc_params,
)
def _edge_kernel(kv_hbm, q_hbm, src_hbm, dst_hbm, zv_hbm, zz_hbm,
                 outv_hbm, outz_hbm,
                 srcv0, dstv0, srcv1, dstv1, dzv,
                 kvb0, qb0, kvb1, qb1, msgv, msgz, fscr,
                 accv, accz, sem0, sem1, sems):
    c = lax.axis_index("c")
    s = lax.axis_index("s")
    wid = c * NS + s
    r0 = s * RPT
    z0 = s * ZPT
    pltpu.sync_copy(zv_hbm, accv.at[pl.ds(r0, RPT)])
    pltpu.sync_copy(zz_hbm, accz.at[pl.ds(z0, ZPT)])
    plsc.subcore_barrier()

    lane = lax.iota(jnp.int32, D)
    zvec = jnp.zeros((D,), jnp.float32)

    def zinit_body(e, carry):
        for kk in range(H):
            msgz[e, pl.ds(kk * D, D)] = zvec
        return carry

    lax.fori_loop(0, B, zinit_body, 0)

    ebase = wid * EWP

    slots = ((srcv0, dstv0, kvb0, qb0, sem0),
             (srcv1, dstv1, kvb1, qb1, sem1))

    def issue(b, sl):
        sv, dv, kb, qbuf, sm_ = sl
        off = ebase + b * B
        pltpu.sync_copy(src_hbm.at[pl.ds(off, B)], sv)
        pltpu.sync_copy(dst_hbm.at[pl.ds(off, B)], dv)
        pltpu.async_copy(kv_hbm.at[sv], kb, sm_)
        pltpu.async_copy(q_hbm.at[dv], qbuf, sm_)

    def wait_slot(sl):
        sv, dv, kb, qbuf, sm_ = sl
        pltpu.make_async_copy(kv_hbm.at[sv], kb, sm_).wait()
        pltpu.make_async_copy(q_hbm.at[dv], qbuf, sm_).wait()

    def compute(b, dstv, kvb, qb):
        @pl.when(b > 0)
        def _():
            pltpu.make_async_copy(msgv, accv.at[dstv], sems).wait()

        def pair_body(p, carry2):
            es = tuple(8 * p + t for t in range(8))
            xs = []
            for e in es:
                prods = [kvb[e, pl.ds(i * D, D)] * qb[e, pl.ds(i * D, D)]
                         for i in range(H)]
                while len(prods) > 1:
                    prods = [prods[i] + prods[i + 1]
                             for i in range(0, len(prods), 2)]
                xs.append(prods[0])
            sms = []
            for x in xs:
                s = x + lax.rev(x, (0,))
                s = jnp.exp(jnp.minimum(jnp.maximum(s, -5.0), 5.0))
                sms.append(jnp.where(lane < H, s, 0.0))
            wrps = []
            for e, sm in zip(es, sms):
                fscr[e, pl.ds(0, D)] = sm
                wrps.append(sm + lax.rev(sm, (0,)))
            vss = [[kvb[e, pl.ds(HD + i * D, D)] for i in range(H)]
                   for e in es]
            for e, wrp, vs in zip(es, wrps, vss):
                for i in range(H):
                    msgv[e, pl.ds(i * D, D)] = wrp * vs[i]
            return carry2

        lax.fori_loop(0, B // 8, pair_body, 0)

        def zgrp_body(g, carry2):
            e16 = lane + g * D
            d16 = dstv[pl.ds(g * D, D)]
            dzv[pl.ds(g * D, D)] = lax.shift_right_logical(d16, 4)
            col = lax.shift_left(d16 & 15, 3)
            whs = [plsc.load_gather(fscr, [e16, jnp.full((D,), h, jnp.int32)])
                   for h in range(H)]
            for h in range(H):
                plsc.store_scatter(msgz, [e16, col + h], whs[h])
            return carry2

        lax.fori_loop(0, B // D, zgrp_body, 0)
        pltpu.async_copy(msgv, accv.at[dstv], sems, add=True)
        pltpu.sync_copy(msgz, accz.at[dzv], add=True)

        def zclr_body(g, carry2):
            e16 = lane + g * D
            d16 = dstv[pl.ds(g * D, D)]
            col = lax.shift_left(d16 & 15, 3)
            for h in range(H):
                plsc.store_scatter(msgz, [e16, col + h], zvec)
            return carry2

        lax.fori_loop(0, B // D, zclr_body, 0)

    issue(0, slots[0])

    def super_body(j, carry):
        for k in (0, 1):
            b = 2 * j + k
            wait_slot(slots[k])

            @pl.when(b + 1 < NBLK)
            def _():
                issue(b + 1, slots[1 - k])

            compute(b, slots[k][1], slots[k][2], slots[k][3])
        return carry

    lax.fori_loop(0, NBLK // 2, super_body, 0)
    pltpu.make_async_copy(msgv, accv.at[dstv1], sems).wait()
    plsc.subcore_barrier()
    pltpu.sync_copy(accv.at[pl.ds(r0, RPT)], outv_hbm.at[pl.ds(c * N_PAD + r0, RPT)])
    pltpu.sync_copy(accz.at[pl.ds(z0, ZPT)], outz_hbm.at[pl.ds(c * ZROWS + z0, ZPT)])


@functools.partial(
    pl.kernel,
    out_type=jax.ShapeDtypeStruct((N_PAD, HD), jnp.float32),
    mesh=_mesh,
    scratch_types=[
        pltpu.VMEM((CH, HD), jnp.float32),
        pltpu.VMEM((CH, HD), jnp.float32),
        pltpu.VMEM((CH // D + 1, HD), jnp.float32),
        pltpu.VMEM((CH // D + 1, HD), jnp.float32),
        pltpu.VMEM((CH, HD), jnp.float32),
    ],
    compiler_params=_sc_params,
)
def _combine_kernel(partv_hbm, partz_hbm, out_hbm, va, vb, za, zb, ob):
    lane = lax.iota(jnp.int32, D)
    c = lax.axis_index("c")
    s = lax.axis_index("s")
    wid = c * NS + s
    niter = (NCHK + NW - 1) // NW

    def iter_body(j, carry):
        chunk = wid + j * NW

        @pl.when(chunk < NCHK)
        def _():
            row = chunk * CH
            zrow = chunk * (CH // D)
            pltpu.sync_copy(partv_hbm.at[pl.ds(row, CH)], va)
            pltpu.sync_copy(partv_hbm.at[pl.ds(N_PAD + row, CH)], vb)
            pltpu.sync_copy(partz_hbm.at[pl.ds(zrow, CH // D)],
                            za.at[pl.ds(0, CH // D)])
            pltpu.sync_copy(partz_hbm.at[pl.ds(ZROWS + zrow, CH // D)],
                            zb.at[pl.ds(0, CH // D)])

            def row_body(r, carry2):
                zi = lax.shift_right_logical(r, 4)
                zc = lax.shift_left(r & 15, 3)
                zs = za[zi, pl.ds(zc, D)] + zb[zi, pl.ds(zc, D)]
                zm = jnp.where(lane < H, zs, 0.0)
                rz = 1.0 / (zm + lax.rev(zm, (0,)))
                for i in range(H):
                    wv = va[r, pl.ds(i * D, D)] + vb[r, pl.ds(i * D, D)]
                    ob[r, pl.ds(i * D, D)] = wv * rz
                return carry2

            lax.fori_loop(0, CH, row_body, 0)
            pltpu.sync_copy(ob, out_hbm.at[pl.ds(row, CH)])

        return carry

    lax.fori_loop(0, niter, iter_body, 0)


def kernel(h, edge_index, WQ, bQ, WK, bK, WV, bV):
    ei = edge_index.astype(jnp.int32)
    pad = E_PAD - E
    src = jnp.concatenate([ei[0], jnp.zeros((pad,), jnp.int32)])
    dst = jnp.concatenate([ei[1], jnp.full((pad,), PAD_DST, jnp.int32)])
    def _orig_col(j):
        i, r = divmod(j, D)
        h = r if r < H else 15 - r
        d = 2 * i + (1 if r >= H else 0)
        return h * D + d
    perm = jnp.array([_orig_col(j) for j in range(HD)], jnp.int32)
    q, kv = _proj(h, WQ[:, perm], WK[:, perm], WV[:, perm],
                  bQ[perm].reshape(1, HD), bK[perm].reshape(1, HD),
                  bV[perm].reshape(1, HD))
    zv = jnp.zeros((RPT, HD), jnp.float32)
    zz = jnp.zeros((ZPT, HD), jnp.float32)
    partv, partz = _edge_kernel(kv, q, src, dst, zv, zz)
    out = _combine_kernel(partv, partz)
    colidx = jnp.array(
        [8 * d + (h if d % 2 == 0 else 7 - h)
         for h in range(H) for d in range(D)], jnp.int32)
    return out[:N][:, colidx].reshape(N, H, D)


# speedup vs baseline: 1.0612x; 1.0612x over previous
import functools

import jax
import jax.numpy as jnp
from jax import lax
from jax.experimental import pallas as pl
from jax.experimental.pallas import tpu as pltpu
from jax.experimental.pallas import tpu_sc as plsc

N = 10000
E = 320000
IN_DIM = 128
H = 8
D = 16
HD = H * D
KV_W = 2 * HD
ACC_W = 144

NC = 2
NS = 16
NW = NC * NS
EW = E // NW
B = 32
NBLK = 2 * -(-EW // (2 * B))
EWP = NBLK * B
E_PAD = NW * EWP
PAD_DST = 10100
RPT = 632
N_PAD = NS * RPT
ZROWS = 640
ZPT = ZROWS // NS
CH = 128
NCHK = N_PAD // CH

_ROWS_PER_PROJ_BLOCK = 1000


def _proj_body(h_ref, wq_ref, wk_ref, wv_ref, bq_ref, bk_ref, bv_ref,
               q_ref, kv_ref):
    hb = h_ref[...]
    q = jnp.dot(hb, wq_ref[...], preferred_element_type=jnp.float32) + bq_ref[...]
    q_ref[...] = q * 0.25
    kv_ref[:, :HD] = jnp.dot(hb, wk_ref[...], preferred_element_type=jnp.float32) + bk_ref[...]
    kv_ref[:, HD:] = jnp.dot(hb, wv_ref[...], preferred_element_type=jnp.float32) + bv_ref[...]


_proj = pl.pallas_call(
    _proj_body,
    grid=(N // _ROWS_PER_PROJ_BLOCK,),
    in_specs=[
        pl.BlockSpec((_ROWS_PER_PROJ_BLOCK, IN_DIM), lambda i: (i, 0)),
        pl.BlockSpec((IN_DIM, HD), lambda i: (0, 0)),
        pl.BlockSpec((IN_DIM, HD), lambda i: (0, 0)),
        pl.BlockSpec((IN_DIM, HD), lambda i: (0, 0)),
        pl.BlockSpec((1, HD), lambda i: (0, 0)),
        pl.BlockSpec((1, HD), lambda i: (0, 0)),
        pl.BlockSpec((1, HD), lambda i: (0, 0)),
    ],
    out_specs=[
        pl.BlockSpec((_ROWS_PER_PROJ_BLOCK, HD), lambda i: (i, 0)),
        pl.BlockSpec((_ROWS_PER_PROJ_BLOCK, KV_W), lambda i: (i, 0)),
    ],
    out_shape=[
        jax.ShapeDtypeStruct((N, HD), jnp.float32),
        jax.ShapeDtypeStruct((N, KV_W), jnp.float32),
    ],
)

_mesh = plsc.VectorSubcoreMesh(core_axis_name="c", subcore_axis_name="s")
_sc_params = pltpu.CompilerParams(needs_layout_passes=False)


@functools.partial(
    pl.kernel,
    out_type=(
        jax.ShapeDtypeStruct((NC * N_PAD, HD), jnp.float32),
        jax.ShapeDtypeStruct((NC * ZROWS, HD), jnp.float32),
    ),
    mesh=_mesh,
    scratch_types=[
        pltpu.VMEM((B,), jnp.int32),
        pltpu.VMEM((B,), jnp.int32),
        pltpu.VMEM((B,), jnp.int32),
        pltpu.VMEM((B,), jnp.int32),
        pltpu.VMEM((B,), jnp.int32),
        pltpu.VMEM((B, KV_W), jnp.float32),
        pltpu.VMEM((B, HD), jnp.float32),
        pltpu.VMEM((B, KV_W), jnp.float32),
        pltpu.VMEM((B, HD), jnp.float32),
        pltpu.VMEM((B, HD), jnp.float32),
        pltpu.VMEM((B, HD), jnp.float32),
        pltpu.VMEM((B, D), jnp.float32),
        pltpu.VMEM_SHARED((N_PAD, HD), jnp.float32),
        pltpu.VMEM_SHARED((ZROWS, HD), jnp.float32),
        pltpu.SemaphoreType.DMA,
        pltpu.SemaphoreType.DMA,
        pltpu.SemaphoreType.DMA,
    ],
    compiler_params=_sc_params,
)
def _edge_kernel(kv_hbm, q_hbm, src_hbm, dst_hbm, zv_hbm, zz_hbm,
                 outv_hbm, outz_hbm,
                 srcv0, dstv0, srcv1, dstv1, dzv,
                 kvb0, qb0, kvb1, qb1, msgv, msgz, fscr,
                 accv, accz, sem0, sem1, sems):
    c = lax.axis_index("c")
    s = lax.axis_index("s")
    wid = c * NS + s
    r0 = s * RPT
    z0 = s * ZPT
    pltpu.sync_copy(zv_hbm, accv.at[pl.ds(r0, RPT)])
    pltpu.sync_copy(zz_hbm, accz.at[pl.ds(z0, ZPT)])
    plsc.subcore_barrier()

    lane = lax.iota(jnp.int32, D)
    zvec = jnp.zeros((D,), jnp.float32)

    def zinit_body(e, carry):
        for kk in range(H):
            msgz[e, pl.ds(kk * D, D)] = zvec
        return carry

    lax.fori_loop(0, B, zinit_body, 0)

    ebase = wid * EWP

    slots = ((srcv0, dstv0, kvb0, qb0, sem0),
             (srcv1, dstv1, kvb1, qb1, sem1))

    def issue(b, sl):
        sv, dv, kb, qbuf, sm_ = sl
        off = ebase + b * B
        pltpu.sync_copy(src_hbm.at[pl.ds(off, B)], sv)
        pltpu.sync_copy(dst_hbm.at[pl.ds(off, B)], dv)
        pltpu.async_copy(kv_hbm.at[sv], kb, sm_)
        pltpu.async_copy(q_hbm.at[dv], qbuf, sm_)

    def wait_slot(sl):
        sv, dv, kb, qbuf, sm_ = sl
        pltpu.make_async_copy(kv_hbm.at[sv], kb, sm_).wait()
        pltpu.make_async_copy(q_hbm.at[dv], qbuf, sm_).wait()

    def compute(b, dstv, kvb, qb):
        @pl.when(b > 0)
        def _():
            pltpu.make_async_copy(msgv, accv.at[dstv], sems).wait()

        @plsc.parallel_loop(0, B, 1, unroll=4)
        def _pl(e):
            prods = [kvb[e, pl.ds(i * D, D)] * qb[e, pl.ds(i * D, D)]
                     for i in range(H)]
            while len(prods) > 1:
                prods = [prods[i] + prods[i + 1]
                         for i in range(0, len(prods), 2)]
            x = prods[0]
            s = x + lax.rev(x, (0,))
            s = jnp.exp(jnp.minimum(jnp.maximum(s, -5.0), 5.0))
            sm = jnp.where(lane < H, s, 0.0)
            fscr[e, pl.ds(0, D)] = sm
            wrp = sm + lax.rev(sm, (0,))
            vs = [kvb[e, pl.ds(HD + i * D, D)] for i in range(H)]
            for i in range(H):
                msgv[e, pl.ds(i * D, D)] = wrp * vs[i]

        def zgrp_body(g, carry2):
            e16 = lane + g * D
            d16 = dstv[pl.ds(g * D, D)]
            dzv[pl.ds(g * D, D)] = lax.shift_right_logical(d16, 4)
            col = lax.shift_left(d16 & 15, 3)
            whs = [plsc.load_gather(fscr, [e16, jnp.full((D,), h, jnp.int32)])
                   for h in range(H)]
            for h in range(H):
                plsc.store_scatter(msgz, [e16, col + h], whs[h])
            return carry2

        lax.fori_loop(0, B // D, zgrp_body, 0)
        pltpu.async_copy(msgv, accv.at[dstv], sems, add=True)
        pltpu.sync_copy(msgz, accz.at[dzv], add=True)

        def zclr_body(g, carry2):
            e16 = lane + g * D
            d16 = dstv[pl.ds(g * D, D)]
            col = lax.shift_left(d16 & 15, 3)
            for h in range(H):
                plsc.store_scatter(msgz, [e16, col + h], zvec)
            return carry2

        lax.fori_loop(0, B // D, zclr_body, 0)

    issue(0, slots[0])

    def super_body(j, carry):
        for k in (0, 1):
            b = 2 * j + k
            wait_slot(slots[k])

            @pl.when(b + 1 < NBLK)
            def _():
                issue(b + 1, slots[1 - k])

            compute(b, slots[k][1], slots[k][2], slots[k][3])
        return carry

    lax.fori_loop(0, NBLK // 2, super_body, 0)
    pltpu.make_async_copy(msgv, accv.at[dstv1], sems).wait()
    plsc.subcore_barrier()
    pltpu.sync_copy(accv.at[pl.ds(r0, RPT)], outv_hbm.at[pl.ds(c * N_PAD + r0, RPT)])
    pltpu.sync_copy(accz.at[pl.ds(z0, ZPT)], outz_hbm.at[pl.ds(c * ZROWS + z0, ZPT)])


@functools.partial(
    pl.kernel,
    out_type=jax.ShapeDtypeStruct((N_PAD, HD), jnp.float32),
    mesh=_mesh,
    scratch_types=[
        pltpu.VMEM((CH, HD), jnp.float32),
        pltpu.VMEM((CH, HD), jnp.float32),
        pltpu.VMEM((CH // D + 1, HD), jnp.float32),
        pltpu.VMEM((CH // D + 1, HD), jnp.float32),
        pltpu.VMEM((CH, HD), jnp.float32),
    ],
    compiler_params=_sc_params,
)
def _combine_kernel(partv_hbm, partz_hbm, out_hbm, va, vb, za, zb, ob):
    lane = lax.iota(jnp.int32, D)
    c = lax.axis_index("c")
    s = lax.axis_index("s")
    wid = c * NS + s
    niter = (NCHK + NW - 1) // NW

    def iter_body(j, carry):
        chunk = wid + j * NW

        @pl.when(chunk < NCHK)
        def _():
            row = chunk * CH
            zrow = chunk * (CH // D)
            pltpu.sync_copy(partv_hbm.at[pl.ds(row, CH)], va)
            pltpu.sync_copy(partv_hbm.at[pl.ds(N_PAD + row, CH)], vb)
            pltpu.sync_copy(partz_hbm.at[pl.ds(zrow, CH // D)],
                            za.at[pl.ds(0, CH // D)])
            pltpu.sync_copy(partz_hbm.at[pl.ds(ZROWS + zrow, CH // D)],
                            zb.at[pl.ds(0, CH // D)])

            def row_body(r, carry2):
                zi = lax.shift_right_logical(r, 4)
                zc = lax.shift_left(r & 15, 3)
                zs = za[zi, pl.ds(zc, D)] + zb[zi, pl.ds(zc, D)]
                zm = jnp.where(lane < H, zs, 0.0)
                rz = 1.0 / (zm + lax.rev(zm, (0,)))
                for i in range(H):
                    wv = va[r, pl.ds(i * D, D)] + vb[r, pl.ds(i * D, D)]
                    ob[r, pl.ds(i * D, D)] = wv * rz
                return carry2

            lax.fori_loop(0, CH, row_body, 0)
            pltpu.sync_copy(ob, out_hbm.at[pl.ds(row, CH)])

        return carry

    lax.fori_loop(0, niter, iter_body, 0)


def kernel(h, edge_index, WQ, bQ, WK, bK, WV, bV):
    ei = edge_index.astype(jnp.int32)
    pad = E_PAD - E
    src = jnp.concatenate([ei[0], jnp.zeros((pad,), jnp.int32)])
    dst = jnp.concatenate([ei[1], jnp.full((pad,), PAD_DST, jnp.int32)])
    def _orig_col(j):
        i, r = divmod(j, D)
        h = r if r < H else 15 - r
        d = 2 * i + (1 if r >= H else 0)
        return h * D + d
    perm = jnp.array([_orig_col(j) for j in range(HD)], jnp.int32)
    q, kv = _proj(h, WQ[:, perm], WK[:, perm], WV[:, perm],
                  bQ[perm].reshape(1, HD), bK[perm].reshape(1, HD),
                  bV[perm].reshape(1, HD))
    zv = jnp.zeros((RPT, HD), jnp.float32)
    zz = jnp.zeros((ZPT, HD), jnp.float32)
    partv, partz = _edge_kernel(kv, q, src, dst, zv, zz)
    out = _combine_kernel(partv, partz)
    colidx = jnp.array(
        [8 * d + (h if d % 2 == 0 else 7 - h)
         for h in range(H) for d in range(D)], jnp.int32)
    return out[:N][:, colidx].reshape(N, H, D)


# speedup vs baseline: 1.0795x; 1.0172x over previous
import functools

import jax
import jax.numpy as jnp
from jax import lax
from jax.experimental import pallas as pl
from jax.experimental.pallas import tpu as pltpu
from jax.experimental.pallas import tpu_sc as plsc

N = 10000
E = 320000
IN_DIM = 128
H = 8
D = 16
HD = H * D
KV_W = 2 * HD
ACC_W = 144

NC = 2
NS = 16
NW = NC * NS
EW = E // NW
B = 32
NBLK = 2 * -(-EW // (2 * B))
EWP = NBLK * B
E_PAD = NW * EWP
PAD_DST = 10100
RPT = 632
N_PAD = NS * RPT
ZROWS = 640
ZPT = ZROWS // NS
CH = 128
NCHK = N_PAD // CH

_ROWS_PER_PROJ_BLOCK = 1000


def _proj_body(h_ref, wq_ref, wk_ref, wv_ref, bq_ref, bk_ref, bv_ref,
               q_ref, kv_ref):
    hb = h_ref[...]
    q = jnp.dot(hb, wq_ref[...], preferred_element_type=jnp.float32) + bq_ref[...]
    q_ref[...] = q * 0.25
    kv_ref[:, :HD] = jnp.dot(hb, wk_ref[...], preferred_element_type=jnp.float32) + bk_ref[...]
    kv_ref[:, HD:] = jnp.dot(hb, wv_ref[...], preferred_element_type=jnp.float32) + bv_ref[...]


_proj = pl.pallas_call(
    _proj_body,
    grid=(N // _ROWS_PER_PROJ_BLOCK,),
    in_specs=[
        pl.BlockSpec((_ROWS_PER_PROJ_BLOCK, IN_DIM), lambda i: (i, 0)),
        pl.BlockSpec((IN_DIM, HD), lambda i: (0, 0)),
        pl.BlockSpec((IN_DIM, HD), lambda i: (0, 0)),
        pl.BlockSpec((IN_DIM, HD), lambda i: (0, 0)),
        pl.BlockSpec((1, HD), lambda i: (0, 0)),
        pl.BlockSpec((1, HD), lambda i: (0, 0)),
        pl.BlockSpec((1, HD), lambda i: (0, 0)),
    ],
    out_specs=[
        pl.BlockSpec((_ROWS_PER_PROJ_BLOCK, HD), lambda i: (i, 0)),
        pl.BlockSpec((_ROWS_PER_PROJ_BLOCK, KV_W), lambda i: (i, 0)),
    ],
    out_shape=[
        jax.ShapeDtypeStruct((N, HD), jnp.float32),
        jax.ShapeDtypeStruct((N, KV_W), jnp.float32),
    ],
)

_mesh = plsc.VectorSubcoreMesh(core_axis_name="c", subcore_axis_name="s")
_sc_params = pltpu.CompilerParams(needs_layout_passes=False)


@functools.partial(
    pl.kernel,
    out_type=(
        jax.ShapeDtypeStruct((NC * N_PAD, HD), jnp.float32),
        jax.ShapeDtypeStruct((NC * ZROWS, HD), jnp.float32),
    ),
    mesh=_mesh,
    scratch_types=[
        pltpu.VMEM((B,), jnp.int32),
        pltpu.VMEM((B,), jnp.int32),
        pltpu.VMEM((B,), jnp.int32),
        pltpu.VMEM((B,), jnp.int32),
        pltpu.VMEM((B,), jnp.int32),
        pltpu.VMEM((B, KV_W), jnp.float32),
        pltpu.VMEM((B, HD), jnp.float32),
        pltpu.VMEM((B, KV_W), jnp.float32),
        pltpu.VMEM((B, HD), jnp.float32),
        pltpu.VMEM((B, HD), jnp.float32),
        pltpu.VMEM((B, HD), jnp.float32),
        pltpu.VMEM((B, D), jnp.float32),
        pltpu.VMEM_SHARED((N_PAD, HD), jnp.float32),
        pltpu.VMEM_SHARED((ZROWS, HD), jnp.float32),
        pltpu.SemaphoreType.DMA,
        pltpu.SemaphoreType.DMA,
        pltpu.SemaphoreType.DMA,
    ],
    compiler_params=_sc_params,
)
def _edge_kernel(kv_hbm, q_hbm, src_hbm, dst_hbm, zv_hbm, zz_hbm,
                 outv_hbm, outz_hbm,
                 srcv0, dstv0, srcv1, dstv1, dzv,
                 kvb0, qb0, kvb1, qb1, msgv, msgz, fscr,
                 accv, accz, sem0, sem1, sems):
    c = lax.axis_index("c")
    s = lax.axis_index("s")
    wid = c * NS + s
    r0 = s * RPT
    z0 = s * ZPT
    pltpu.sync_copy(zv_hbm, accv.at[pl.ds(r0, RPT)])
    pltpu.sync_copy(zz_hbm, accz.at[pl.ds(z0, ZPT)])
    plsc.subcore_barrier()

    lane = lax.iota(jnp.int32, D)
    zvec = jnp.zeros((D,), jnp.float32)

    def zinit_body(e, carry):
        for kk in range(H):
            msgz[e, pl.ds(kk * D, D)] = zvec
        return carry

    lax.fori_loop(0, B, zinit_body, 0)

    ebase = wid * EWP

    slots = ((srcv0, dstv0, kvb0, qb0, sem0),
             (srcv1, dstv1, kvb1, qb1, sem1))

    def issue(b, sl):
        sv, dv, kb, qbuf, sm_ = sl
        off = ebase + b * B
        pltpu.sync_copy(src_hbm.at[pl.ds(off, B)], sv)
        pltpu.sync_copy(dst_hbm.at[pl.ds(off, B)], dv)
        pltpu.async_copy(kv_hbm.at[sv], kb, sm_)
        pltpu.async_copy(q_hbm.at[dv], qbuf, sm_)

    def wait_slot(sl):
        sv, dv, kb, qbuf, sm_ = sl
        pltpu.make_async_copy(kv_hbm.at[sv], kb, sm_).wait()
        pltpu.make_async_copy(q_hbm.at[dv], qbuf, sm_).wait()

    def compute(b, dstv, kvb, qb):
        @pl.when(b > 0)
        def _():
            pltpu.make_async_copy(msgv, accv.at[dstv], sems).wait()

        @plsc.parallel_loop(0, B, 1, unroll=8)
        def _pl(e):
            prods = [kvb[e, pl.ds(i * D, D)] * qb[e, pl.ds(i * D, D)]
                     for i in range(H)]
            while len(prods) > 1:
                prods = [prods[i] + prods[i + 1]
                         for i in range(0, len(prods), 2)]
            x = prods[0]
            s = x + lax.rev(x, (0,))
            s = jnp.exp(jnp.minimum(jnp.maximum(s, -5.0), 5.0))
            sm = jnp.where(lane < H, s, 0.0)
            fscr[e, pl.ds(0, D)] = sm
            wrp = sm + lax.rev(sm, (0,))
            vs = [kvb[e, pl.ds(HD + i * D, D)] for i in range(H)]
            for i in range(H):
                msgv[e, pl.ds(i * D, D)] = wrp * vs[i]

        @plsc.parallel_loop(0, B // D, 1, unroll=2)
        def _zg(g):
            e16 = lane + g * D
            d16 = dstv[pl.ds(g * D, D)]
            dzv[pl.ds(g * D, D)] = lax.shift_right_logical(d16, 4)
            col = lax.shift_left(d16 & 15, 3)
            whs = [plsc.load_gather(fscr, [e16, jnp.full((D,), h, jnp.int32)])
                   for h in range(H)]
            for h in range(H):
                plsc.store_scatter(msgz, [e16, col + h], whs[h])
        pltpu.async_copy(msgv, accv.at[dstv], sems, add=True)
        pltpu.sync_copy(msgz, accz.at[dzv], add=True)

        @plsc.parallel_loop(0, B // D, 1, unroll=2)
        def _zc(g):
            e16 = lane + g * D
            d16 = dstv[pl.ds(g * D, D)]
            col = lax.shift_left(d16 & 15, 3)
            for h in range(H):
                plsc.store_scatter(msgz, [e16, col + h], zvec)

    issue(0, slots[0])

    def super_body(j, carry):
        for k in (0, 1):
            b = 2 * j + k
            wait_slot(slots[k])

            @pl.when(b + 1 < NBLK)
            def _():
                issue(b + 1, slots[1 - k])

            compute(b, slots[k][1], slots[k][2], slots[k][3])
        return carry

    lax.fori_loop(0, NBLK // 2, super_body, 0)
    pltpu.make_async_copy(msgv, accv.at[dstv1], sems).wait()
    plsc.subcore_barrier()
    pltpu.sync_copy(accv.at[pl.ds(r0, RPT)], outv_hbm.at[pl.ds(c * N_PAD + r0, RPT)])
    pltpu.sync_copy(accz.at[pl.ds(z0, ZPT)], outz_hbm.at[pl.ds(c * ZROWS + z0, ZPT)])


@functools.partial(
    pl.kernel,
    out_type=jax.ShapeDtypeStruct((N_PAD, HD), jnp.float32),
    mesh=_mesh,
    scratch_types=[
        pltpu.VMEM((CH, HD), jnp.float32),
        pltpu.VMEM((CH, HD), jnp.float32),
        pltpu.VMEM((CH // D + 1, HD), jnp.float32),
        pltpu.VMEM((CH // D + 1, HD), jnp.float32),
        pltpu.VMEM((CH, HD), jnp.float32),
    ],
    compiler_params=_sc_params,
)
def _combine_kernel(partv_hbm, partz_hbm, out_hbm, va, vb, za, zb, ob):
    lane = lax.iota(jnp.int32, D)
    c = lax.axis_index("c")
    s = lax.axis_index("s")
    wid = c * NS + s
    niter = (NCHK + NW - 1) // NW

    def iter_body(j, carry):
        chunk = wid + j * NW

        @pl.when(chunk < NCHK)
        def _():
            row = chunk * CH
            zrow = chunk * (CH // D)
            pltpu.sync_copy(partv_hbm.at[pl.ds(row, CH)], va)
            pltpu.sync_copy(partv_hbm.at[pl.ds(N_PAD + row, CH)], vb)
            pltpu.sync_copy(partz_hbm.at[pl.ds(zrow, CH // D)],
                            za.at[pl.ds(0, CH // D)])
            pltpu.sync_copy(partz_hbm.at[pl.ds(ZROWS + zrow, CH // D)],
                            zb.at[pl.ds(0, CH // D)])

            def row_body(r, carry2):
                zi = lax.shift_right_logical(r, 4)
                zc = lax.shift_left(r & 15, 3)
                zs = za[zi, pl.ds(zc, D)] + zb[zi, pl.ds(zc, D)]
                zm = jnp.where(lane < H, zs, 0.0)
                rz = 1.0 / (zm + lax.rev(zm, (0,)))
                for i in range(H):
                    wv = va[r, pl.ds(i * D, D)] + vb[r, pl.ds(i * D, D)]
                    ob[r, pl.ds(i * D, D)] = wv * rz
                return carry2

            lax.fori_loop(0, CH, row_body, 0)
            pltpu.sync_copy(ob, out_hbm.at[pl.ds(row, CH)])

        return carry

    lax.fori_loop(0, niter, iter_body, 0)


def kernel(h, edge_index, WQ, bQ, WK, bK, WV, bV):
    ei = edge_index.astype(jnp.int32)
    pad = E_PAD - E
    src = jnp.concatenate([ei[0], jnp.zeros((pad,), jnp.int32)])
    dst = jnp.concatenate([ei[1], jnp.full((pad,), PAD_DST, jnp.int32)])
    def _orig_col(j):
        i, r = divmod(j, D)
        h = r if r < H else 15 - r
        d = 2 * i + (1 if r >= H else 0)
        return h * D + d
    perm = jnp.array([_orig_col(j) for j in range(HD)], jnp.int32)
    q, kv = _proj(h, WQ[:, perm], WK[:, perm], WV[:, perm],
                  bQ[perm].reshape(1, HD), bK[perm].reshape(1, HD),
                  bV[perm].reshape(1, HD))
    zv = jnp.zeros((RPT, HD), jnp.float32)
    zz = jnp.zeros((ZPT, HD), jnp.float32)
    partv, partz = _edge_kernel(kv, q, src, dst, zv, zz)
    out = _combine_kernel(partv, partz)
    colidx = jnp.array(
        [8 * d + (h if d % 2 == 0 else 7 - h)
         for h in range(H) for d in range(D)], jnp.int32)
    return out[:N][:, colidx].reshape(N, H, D)


# speedup vs baseline: 1.2914x; 1.1963x over previous
import functools

import jax
import jax.numpy as jnp
from jax import lax
from jax.experimental import pallas as pl
from jax.experimental.pallas import tpu as pltpu
from jax.experimental.pallas import tpu_sc as plsc

N = 10000
E = 320000
IN_DIM = 128
H = 8
D = 16
HD = H * D
KV_W = 2 * HD
ACC_W = 144

NC = 2
NS = 16
NW = NC * NS
EW = E // NW
B = 32
NBLK = 4 * -(-EW // (4 * B))
EWP = NBLK * B
E_PAD = NW * EWP
PAD_DST = 10100
RPT = 632
N_PAD = NS * RPT
ZROWS = 640
ZPT = ZROWS // NS
CH = 128
NCHK = N_PAD // CH

_ROWS_PER_PROJ_BLOCK = 1000


def _proj_body(h_ref, wq_ref, wk_ref, wv_ref, bq_ref, bk_ref, bv_ref,
               q_ref, kv_ref):
    hb = h_ref[...]
    q = jnp.dot(hb, wq_ref[...], preferred_element_type=jnp.float32) + bq_ref[...]
    q_ref[...] = q * 0.25
    kv_ref[:, :HD] = jnp.dot(hb, wk_ref[...], preferred_element_type=jnp.float32) + bk_ref[...]
    kv_ref[:, HD:] = jnp.dot(hb, wv_ref[...], preferred_element_type=jnp.float32) + bv_ref[...]


_proj = pl.pallas_call(
    _proj_body,
    grid=(N // _ROWS_PER_PROJ_BLOCK,),
    in_specs=[
        pl.BlockSpec((_ROWS_PER_PROJ_BLOCK, IN_DIM), lambda i: (i, 0)),
        pl.BlockSpec((IN_DIM, HD), lambda i: (0, 0)),
        pl.BlockSpec((IN_DIM, HD), lambda i: (0, 0)),
        pl.BlockSpec((IN_DIM, HD), lambda i: (0, 0)),
        pl.BlockSpec((1, HD), lambda i: (0, 0)),
        pl.BlockSpec((1, HD), lambda i: (0, 0)),
        pl.BlockSpec((1, HD), lambda i: (0, 0)),
    ],
    out_specs=[
        pl.BlockSpec((_ROWS_PER_PROJ_BLOCK, HD), lambda i: (i, 0)),
        pl.BlockSpec((_ROWS_PER_PROJ_BLOCK, KV_W), lambda i: (i, 0)),
    ],
    out_shape=[
        jax.ShapeDtypeStruct((N, HD), jnp.float32),
        jax.ShapeDtypeStruct((N, KV_W), jnp.float32),
    ],
)

_mesh = plsc.VectorSubcoreMesh(core_axis_name="c", subcore_axis_name="s")
_sc_params = pltpu.CompilerParams(needs_layout_passes=False)


@functools.partial(
    pl.kernel,
    out_type=(
        jax.ShapeDtypeStruct((NC * N_PAD, HD), jnp.float32),
        jax.ShapeDtypeStruct((NC * ZROWS, HD), jnp.float32),
    ),
    mesh=_mesh,
    scratch_types=[
        pltpu.VMEM((4, B), jnp.int32),
        pltpu.VMEM((4, B), jnp.int32),
        pltpu.VMEM((B,), jnp.int32),
        pltpu.VMEM((B, KV_W), jnp.float32),
        pltpu.VMEM((B, HD), jnp.float32),
        pltpu.VMEM((B, KV_W), jnp.float32),
        pltpu.VMEM((B, HD), jnp.float32),
        pltpu.VMEM((B, HD), jnp.float32),
        pltpu.VMEM((B, HD), jnp.float32),
        pltpu.VMEM((B, D), jnp.float32),
        pltpu.VMEM_SHARED((N_PAD, HD), jnp.float32),
        pltpu.VMEM_SHARED((ZROWS, HD), jnp.float32),
        pltpu.SemaphoreType.DMA,
        pltpu.SemaphoreType.DMA,
        pltpu.SemaphoreType.DMA,
        pltpu.SemaphoreType.DMA,
        pltpu.SemaphoreType.DMA,
        pltpu.SemaphoreType.DMA,
        pltpu.SemaphoreType.DMA,
    ],
    compiler_params=_sc_params,
)
def _edge_kernel(kv_hbm, q_hbm, src_hbm, dst_hbm, zv_hbm, zz_hbm,
                 outv_hbm, outz_hbm,
                 srcv4, dstv4, dzv,
                 kvb0, qb0, kvb1, qb1, msgv, msgz, fscr,
                 accv, accz, semr0, semr1, sems,
                 semi0, semi1, semi2, semi3):
    c = lax.axis_index("c")
    s = lax.axis_index("s")
    wid = c * NS + s
    r0 = s * RPT
    z0 = s * ZPT
    pltpu.sync_copy(zv_hbm, accv.at[pl.ds(r0, RPT)])
    pltpu.sync_copy(zz_hbm, accz.at[pl.ds(z0, ZPT)])
    plsc.subcore_barrier()

    lane = lax.iota(jnp.int32, D)
    zvec = jnp.zeros((D,), jnp.float32)

    def zinit_body(e, carry):
        for kk in range(H):
            msgz[e, pl.ds(kk * D, D)] = zvec
        return carry

    lax.fori_loop(0, B, zinit_body, 0)

    ebase = wid * EWP

    rowslots = ((kvb0, qb0, semr0), (kvb1, qb1, semr1))
    semis = (semi0, semi1, semi2, semi3)

    def idx_refs(i):
        return srcv4.at[i], dstv4.at[i]

    def issue_idx(b, i):
        sv, dv = idx_refs(i)
        off = ebase + b * B
        pltpu.async_copy(src_hbm.at[pl.ds(off, B)], sv, semis[i])
        pltpu.async_copy(dst_hbm.at[pl.ds(off, B)], dv, semis[i])

    def wait_idx(b, i):
        sv, dv = idx_refs(i)
        off = ebase + b * B
        pltpu.make_async_copy(src_hbm.at[pl.ds(off, B)], sv, semis[i]).wait()
        pltpu.make_async_copy(dst_hbm.at[pl.ds(off, B)], dv, semis[i]).wait()

    def issue_rows(i, rs):
        sv, dv = idx_refs(i)
        kb, qbuf, sm_ = rs
        pltpu.async_copy(kv_hbm.at[sv], kb, sm_)
        pltpu.async_copy(q_hbm.at[dv], qbuf, sm_)

    def wait_rows(i, rs):
        sv, dv = idx_refs(i)
        kb, qbuf, sm_ = rs
        pltpu.make_async_copy(kv_hbm.at[sv], kb, sm_).wait()
        pltpu.make_async_copy(q_hbm.at[dv], qbuf, sm_).wait()

    def compute(b, dstv, kvb, qb):
        @plsc.parallel_loop(0, B, 1, unroll=8)
        def _pl(e):
            prods = [kvb[e, pl.ds(i * D, D)] * qb[e, pl.ds(i * D, D)]
                     for i in range(H)]
            while len(prods) > 1:
                prods = [prods[i] + prods[i + 1]
                         for i in range(0, len(prods), 2)]
            x = prods[0]
            s = x + lax.rev(x, (0,))
            s = jnp.exp(jnp.minimum(jnp.maximum(s, -5.0), 5.0))
            sm = jnp.where(lane < H, s, 0.0)
            fscr[e, pl.ds(0, D)] = sm
            wrp = sm + lax.rev(sm, (0,))
            vs = [kvb[e, pl.ds(HD + i * D, D)] for i in range(H)]
            for i in range(H):
                msgv[e, pl.ds(i * D, D)] = wrp * vs[i]

        @plsc.parallel_loop(0, B // D, 1, unroll=2)
        def _zg(g):
            e16 = lane + g * D
            d16 = dstv[pl.ds(g * D, D)]
            dzv[pl.ds(g * D, D)] = lax.shift_right_logical(d16, 4)
            col = lax.shift_left(d16 & 15, 3)
            whs = [plsc.load_gather(fscr, [e16, jnp.full((D,), h, jnp.int32)])
                   for h in range(H)]
            for h in range(H):
                plsc.store_scatter(msgz, [e16, col + h], whs[h])
        pltpu.async_copy(msgv, accv.at[dstv], sems, add=True)
        pltpu.sync_copy(msgz, accz.at[dzv], add=True)

        @plsc.parallel_loop(0, B // D, 1, unroll=2)
        def _zc(g):
            e16 = lane + g * D
            d16 = dstv[pl.ds(g * D, D)]
            col = lax.shift_left(d16 & 15, 3)
            for h in range(H):
                plsc.store_scatter(msgz, [e16, col + h], zvec)

    issue_idx(0, 0)
    issue_idx(1, 1)
    issue_idx(2, 2)
    wait_idx(0, 0)
    issue_rows(0, rowslots[0])

    def super_body(m, carry):
        for t in range(4):
            b = 4 * m + t
            i = t
            k = t & 1
            wait_rows(i, rowslots[k])

            @pl.when(b + 1 < NBLK)
            def _():
                wait_idx(b + 1, (t + 1) & 3)
                issue_rows((t + 1) & 3, rowslots[1 - k])

            @pl.when(b > 0)
            def _():
                pltpu.make_async_copy(msgv, accv.at[srcv4.at[0]], sems).wait()

            @pl.when(b + 3 < NBLK)
            def _():
                issue_idx(b + 3, (t + 3) & 3)

            sv, dv = idx_refs(i)
            compute(b, dv, rowslots[k][0], rowslots[k][1])
        return carry

    lax.fori_loop(0, NBLK // 4, super_body, 0)
    pltpu.make_async_copy(msgv, accv.at[srcv4.at[0]], sems).wait()
    plsc.subcore_barrier()
    pltpu.sync_copy(accv.at[pl.ds(r0, RPT)], outv_hbm.at[pl.ds(c * N_PAD + r0, RPT)])
    pltpu.sync_copy(accz.at[pl.ds(z0, ZPT)], outz_hbm.at[pl.ds(c * ZROWS + z0, ZPT)])


@functools.partial(
    pl.kernel,
    out_type=jax.ShapeDtypeStruct((N_PAD, HD), jnp.float32),
    mesh=_mesh,
    scratch_types=[
        pltpu.VMEM((CH, HD), jnp.float32),
        pltpu.VMEM((CH, HD), jnp.float32),
        pltpu.VMEM((CH // D + 1, HD), jnp.float32),
        pltpu.VMEM((CH // D + 1, HD), jnp.float32),
        pltpu.VMEM((CH, HD), jnp.float32),
    ],
    compiler_params=_sc_params,
)
def _combine_kernel(partv_hbm, partz_hbm, out_hbm, va, vb, za, zb, ob):
    lane = lax.iota(jnp.int32, D)
    c = lax.axis_index("c")
    s = lax.axis_index("s")
    wid = c * NS + s
    niter = (NCHK + NW - 1) // NW

    def iter_body(j, carry):
        chunk = wid + j * NW

        @pl.when(chunk < NCHK)
        def _():
            row = chunk * CH
            zrow = chunk * (CH // D)
            pltpu.sync_copy(partv_hbm.at[pl.ds(row, CH)], va)
            pltpu.sync_copy(partv_hbm.at[pl.ds(N_PAD + row, CH)], vb)
            pltpu.sync_copy(partz_hbm.at[pl.ds(zrow, CH // D)],
                            za.at[pl.ds(0, CH // D)])
            pltpu.sync_copy(partz_hbm.at[pl.ds(ZROWS + zrow, CH // D)],
                            zb.at[pl.ds(0, CH // D)])

            def row_body(r, carry2):
                zi = lax.shift_right_logical(r, 4)
                zc = lax.shift_left(r & 15, 3)
                zs = za[zi, pl.ds(zc, D)] + zb[zi, pl.ds(zc, D)]
                zm = jnp.where(lane < H, zs, 0.0)
                rz = 1.0 / (zm + lax.rev(zm, (0,)))
                for i in range(H):
                    wv = va[r, pl.ds(i * D, D)] + vb[r, pl.ds(i * D, D)]
                    ob[r, pl.ds(i * D, D)] = wv * rz
                return carry2

            lax.fori_loop(0, CH, row_body, 0)
            pltpu.sync_copy(ob, out_hbm.at[pl.ds(row, CH)])

        return carry

    lax.fori_loop(0, niter, iter_body, 0)


def kernel(h, edge_index, WQ, bQ, WK, bK, WV, bV):
    ei = edge_index.astype(jnp.int32)
    pad = E_PAD - E
    src = jnp.concatenate([ei[0], jnp.zeros((pad,), jnp.int32)])
    dst = jnp.concatenate([ei[1], jnp.full((pad,), PAD_DST, jnp.int32)])
    def _orig_col(j):
        i, r = divmod(j, D)
        h = r if r < H else 15 - r
        d = 2 * i + (1 if r >= H else 0)
        return h * D + d
    perm = jnp.array([_orig_col(j) for j in range(HD)], jnp.int32)
    q, kv = _proj(h, WQ[:, perm], WK[:, perm], WV[:, perm],
                  bQ[perm].reshape(1, HD), bK[perm].reshape(1, HD),
                  bV[perm].reshape(1, HD))
    zv = jnp.zeros((RPT, HD), jnp.float32)
    zz = jnp.zeros((ZPT, HD), jnp.float32)
    partv, partz = _edge_kernel(kv, q, src, dst, zv, zz)
    out = _combine_kernel(partv, partz)
    colidx = jnp.array(
        [8 * d + (h if d % 2 == 0 else 7 - h)
         for h in range(H) for d in range(D)], jnp.int32)
    return out[:N][:, colidx].reshape(N, H, D)
